# baseline (speedup 1.0000x reference)
import jax
import jax.numpy as jnp
from jax.experimental import pallas as pl
from jax.experimental.pallas import tpu as pltpu

_NE = 8192
_D = 32
_CC = 0.25
_TOK = 8 * 32 * 32
_BLK = 256
_NBLK = _TOK // _BLK
_HB = 8


def _vq_kernel(idx_ref, x_ref, w_ref, loss_ref, qst_ref, perp_ref, enc_ref,
               cnt_ref, acc_ref):
    i = pl.program_id(0)
    xb = x_ref[0].reshape(_D, _BLK)
    w = w_ref[...]
    idxv = idx_ref[0, 0, :]
    col = jax.lax.broadcasted_iota(jnp.int32, (_BLK, _NE), 1)
    enc = (col == idxv[:, None]).astype(jnp.float32)
    enc_ref[...] = enc
    qT = jax.lax.dot_general(
        w, enc, (((0,), (1,)), ((), ())),
        precision=jax.lax.Precision.HIGHEST,
        preferred_element_type=jnp.float32)
    qst_ref[...] = (xb + (qT - xb)).reshape(1, _D, _HB, 32)

    @pl.when(i == 0)
    def _init():
        cnt_ref[...] = jnp.zeros_like(cnt_ref)
        acc_ref[0] = 0.0

    cnt_ref[...] += jnp.sum(enc, axis=0, keepdims=True)
    acc_ref[0] += jnp.sum((qT - xb) * (qT - xb))

    @pl.when(i == _NBLK - 1)
    def _fin():
        loss_ref[...] = jnp.full((1, 1), _CC * acc_ref[0] / (_TOK * _D),
                                 dtype=jnp.float32)
        p = cnt_ref[...] / _TOK
        perp_ref[...] = jnp.exp(-jnp.sum(p * jnp.log(p + 1e-10),
                                         keepdims=True))


def kernel(inputs, W):
    flat = jnp.transpose(inputs, (0, 2, 3, 1)).reshape(-1, _D)
    distances = (jnp.sum(flat ** 2, axis=1, keepdims=True)
                 + jnp.sum(W ** 2, axis=1)
                 - 2.0 * jnp.matmul(flat, W.T))
    idx = jnp.argmin(distances, axis=1).astype(jnp.int32)
    idx3 = idx.reshape(_NBLK, 1, _BLK)

    loss, qst, perp, enc = pl.pallas_call(
        _vq_kernel,
        grid=(_NBLK,),
        in_specs=[
            pl.BlockSpec((1, 1, _BLK), lambda i: (i, 0, 0)),
            pl.BlockSpec((1, _D, _HB, 32), lambda i: (i // 4, 0, i % 4, 0)),
            pl.BlockSpec((_NE, _D), lambda i: (0, 0)),
        ],
        out_specs=[
            pl.BlockSpec((1, 1), lambda i: (0, 0)),
            pl.BlockSpec((1, _D, _HB, 32), lambda i: (i // 4, 0, i % 4, 0)),
            pl.BlockSpec((1, 1), lambda i: (0, 0)),
            pl.BlockSpec((_BLK, _NE), lambda i: (i, 0)),
        ],
        out_shape=[
            jax.ShapeDtypeStruct((1, 1), jnp.float32),
            jax.ShapeDtypeStruct((8, _D, 32, 32), jnp.float32),
            jax.ShapeDtypeStruct((1, 1), jnp.float32),
            jax.ShapeDtypeStruct((_TOK, _NE), jnp.float32),
        ],
        scratch_shapes=[
            pltpu.VMEM((1, _NE), jnp.float32),
            pltpu.SMEM((1,), jnp.float32),
        ],
    )(idx3, inputs, W)
    return (loss.reshape(()), qst, perp.reshape(()), enc)


# speedup vs baseline: 1.0766x; 1.0766x over previous
import jax
import jax.numpy as jnp
from jax.experimental import pallas as pl
from jax.experimental.pallas import tpu as pltpu

_NE = 8192
_D = 32
_CC = 0.25
_TOK = 8 * 32 * 32
_BLK = 256
_NBLK = _TOK // _BLK
_HB = 8


_CHK = 1024


def _vq_kernel(idx_ref, x_ref, w_ref, loss_ref, qst_ref, perp_ref, enc_ref,
               cnt_ref, acc_ref):
    i = pl.program_id(0)
    xb = x_ref[0].reshape(_D, _BLK)
    idxv = idx_ref[0, 0, :]

    @pl.when(i == 0)
    def _init():
        cnt_ref[...] = jnp.zeros_like(cnt_ref)
        acc_ref[0] = 0.0

    qT = jnp.zeros((_D, _BLK), jnp.float32)
    for c in range(_NE // _CHK):
        lo = c * _CHK
        col = lo + jax.lax.broadcasted_iota(jnp.int32, (_BLK, _CHK), 1)
        enc = (col == idxv[:, None]).astype(jnp.float32)
        enc_ref[:, lo:lo + _CHK] = enc
        cnt_ref[:, lo:lo + _CHK] += jnp.sum(enc, axis=0, keepdims=True)
        qT += jax.lax.dot_general(
            w_ref[lo:lo + _CHK, :], enc, (((0,), (1,)), ((), ())),
            precision=jax.lax.Precision.HIGHEST,
            preferred_element_type=jnp.float32)
    qst_ref[...] = (xb + (qT - xb)).reshape(1, _D, _HB, 32)
    acc_ref[0] += jnp.sum((qT - xb) * (qT - xb))

    @pl.when(i == _NBLK - 1)
    def _fin():
        loss_ref[...] = jnp.full((1, 1), _CC * acc_ref[0] / (_TOK * _D),
                                 dtype=jnp.float32)
        p = cnt_ref[...] / _TOK
        perp_ref[...] = jnp.exp(-jnp.sum(p * jnp.log(p + 1e-10),
                                         keepdims=True))


def kernel(inputs, W):
    flat = jnp.transpose(inputs, (0, 2, 3, 1)).reshape(-1, _D)
    distances = (jnp.sum(flat ** 2, axis=1, keepdims=True)
                 + jnp.sum(W ** 2, axis=1)
                 - 2.0 * jnp.matmul(flat, W.T))
    idx = jnp.argmin(distances, axis=1).astype(jnp.int32)
    idx3 = idx.reshape(_NBLK, 1, _BLK)

    loss, qst, perp, enc = pl.pallas_call(
        _vq_kernel,
        grid=(_NBLK,),
        in_specs=[
            pl.BlockSpec((1, 1, _BLK), lambda i: (i, 0, 0)),
            pl.BlockSpec((1, _D, _HB, 32), lambda i: (i // 4, 0, i % 4, 0)),
            pl.BlockSpec((_NE, _D), lambda i: (0, 0)),
        ],
        out_specs=[
            pl.BlockSpec((1, 1), lambda i: (0, 0)),
            pl.BlockSpec((1, _D, _HB, 32), lambda i: (i // 4, 0, i % 4, 0)),
            pl.BlockSpec((1, 1), lambda i: (0, 0)),
            pl.BlockSpec((_BLK, _NE), lambda i: (i, 0)),
        ],
        out_shape=[
            jax.ShapeDtypeStruct((1, 1), jnp.float32),
            jax.ShapeDtypeStruct((8, _D, 32, 32), jnp.float32),
            jax.ShapeDtypeStruct((1, 1), jnp.float32),
            jax.ShapeDtypeStruct((_TOK, _NE), jnp.float32),
        ],
        scratch_shapes=[
            pltpu.VMEM((1, _NE), jnp.float32),
            pltpu.SMEM((1,), jnp.float32),
        ],
    )(idx3, inputs, W)
    return (loss.reshape(()), qst, perp.reshape(()), enc)
